# baseline (speedup 1.0000x reference)
import functools

import jax
import jax.numpy as jnp
from jax import lax
from jax.experimental import pallas as pl
from jax.experimental.pallas import tpu as pltpu
from jax.experimental.pallas import tpu_sc as plsc

_B = 4096
_F = 26
_D = 64
_FD = _F * _D
_V = 1040000
_TW = 32768
_TBLK = (_V + _TW - 1) // _TW
_QROWS = _TBLK * (_TW // 4)

_NC = 2
_NS = 16
_NW = _NC * _NS
_BPW = _B // _NW
_CHUNK = 128


def _pairs_table(embT, fcT):
    def pack2(hi, lo):
        uh = lax.bitcast_convert_type(hi, jnp.uint32)
        ul = lax.bitcast_convert_type(lo, jnp.uint32)
        u = (uh & jnp.uint32(0xFFFF0000)) | jnp.right_shift(
            ul, jnp.uint32(16))
        return lax.bitcast_convert_type(u, jnp.float32)

    def body(a_ref, fc_ref, o_ref, o2_ref):
        blk = a_ref[...]
        parts = []
        for t in range(_TW // 512):
            a = blk[:, 512 * t:512 * t + 128]
            b = blk[:, 512 * t + 128:512 * t + 256]
            c = blk[:, 512 * t + 256:512 * t + 384]
            d = blk[:, 512 * t + 384:512 * t + 512]
            parts.append(jnp.concatenate(
                [pack2(a.T, b.T), pack2(c.T, d.T)], axis=1))
        o_ref[...] = jnp.concatenate(parts, axis=0)
        o2_ref[...] = fc_ref[...][0]

    return pl.pallas_call(
        body,
        grid=(_TBLK,),
        in_specs=[pl.BlockSpec((_D, _TW), lambda i: (0, i)),
                  pl.BlockSpec((1, _TW), lambda i: (0, i))],
        out_specs=[pl.BlockSpec((_TW // 4, 128), lambda i: (i, 0)),
                   pl.BlockSpec((_TW,), lambda i: (i,))],
        out_shape=[jax.ShapeDtypeStruct((_QROWS, 128), jnp.float32),
                   jax.ShapeDtypeStruct((_V,), jnp.float32)],
    )(embT, fcT)


def _sc_gather(xt3, pairs, fc_flat):
    mesh = plsc.VectorSubcoreMesh(core_axis_name="c", subcore_axis_name="s")

    @functools.partial(
        pl.kernel,
        mesh=mesh,
        out_type=[jax.ShapeDtypeStruct((_F, _B, 128), jnp.float32),
                  jax.ShapeDtypeStruct((_B,), jnp.float32)],
        scratch_types=[
            pltpu.VMEM((_F, _CHUNK), jnp.int32),
            pltpu.VMEM((_F, _CHUNK), jnp.int32),
            pltpu.VMEM((2, _CHUNK, 128), jnp.float32),
            pltpu.VMEM((_F, _BPW), jnp.float32),
            pltpu.VMEM((_BPW,), jnp.float32),
            pltpu.SemaphoreType.DMA,
            pltpu.SemaphoreType.DMA,
        ],
    )
    def body(xt_hbm, tab_hbm, fc_hbm, rows_out, fc_out,
             idx_v, idxp_v, pairbuf, fcbuf, accv, gsem, fsem):
        wid = lax.axis_index("s") * _NC + lax.axis_index("c")
        pltpu.sync_copy(xt_hbm.at[:, wid], idx_v)
        b0 = wid * _BPW

        def prep_step(j, carry):
            pltpu.async_copy(fc_hbm.at[idx_v.at[j]], fcbuf.at[j], fsem)
            for g in range(_CHUNK // 16):
                v = idx_v[j, pl.ds(g * 16, 16)]
                p = lax.bitwise_or(
                    lax.shift_left(lax.shift_right_logical(v, 9), 7),
                    lax.bitwise_and(v, 127))
                idxp_v[j, pl.ds(g * 16, 16)] = p
            return carry

        lax.fori_loop(0, _F, prep_step, 0)

        pltpu.async_copy(tab_hbm.at[idxp_v.at[0]], pairbuf.at[0], gsem)

        def chunk_step(j, carry):
            slot = lax.rem(j, 2)

            @pl.when(j + 1 < _F)
            def _prefetch():
                pltpu.async_copy(tab_hbm.at[idxp_v.at[j + 1]],
                                 pairbuf.at[lax.rem(j + 1, 2)], gsem)

            pltpu.make_async_copy(tab_hbm.at[idxp_v.at[j]],
                                  pairbuf.at[slot], gsem).wait()
            pltpu.sync_copy(pairbuf.at[slot],
                            rows_out.at[j, pl.ds(b0, _CHUNK)])
            return carry

        lax.fori_loop(0, _F, chunk_step, 0)

        def fc_drain(j, carry):
            pltpu.make_async_copy(fc_hbm.at[idx_v.at[j]], fcbuf.at[j],
                                  fsem).wait()
            return carry

        lax.fori_loop(0, _F, fc_drain, 0)

        for g in range(_BPW // 16):
            acc = jnp.zeros((16,), jnp.float32)
            for f in range(_F):
                acc = acc + fcbuf[f, pl.ds(g * 16, 16)]
            accv[pl.ds(g * 16, 16)] = acc
        pltpu.sync_copy(accv, fc_out.at[pl.ds(wid * _BPW, _BPW)])

    return body(xt3, pairs, fc_flat)


def _tc_fused(rows3, xpad, fcsum, bias, W0, b0, W1, b1, W2, b2, W3, b3):
    bb = 512
    grid = (_B // bb,)

    def body(p_ref, xp_ref, fcsum_ref, bias_ref, b3s_ref,
             w0_ref, b0_ref, w1_ref, b1_ref, w2_ref, b2_ref, w3_ref, o_ref):
        xp = xp_ref[...]
        efs = []
        s = jnp.zeros((bb, _D), jnp.float32)
        sq = jnp.zeros((bb, _D), jnp.float32)
        for f in range(_F):
            xv = xp[:, f:f + 1]
            h2 = (lax.shift_right_logical(xv, 8) & 1) == 1
            sub = (lax.shift_right_logical(xv, 7) & 1) == 1
            pf = p_ref[f]
            pfh = jnp.where(h2, pf[:, _D:], pf[:, :_D])
            u = lax.bitcast_convert_type(pfh, jnp.uint32)
            su = jnp.where(sub, jnp.left_shift(u, jnp.uint32(16)),
                           u & jnp.uint32(0xFFFF0000))
            ef = lax.bitcast_convert_type(su, jnp.float32)
            efs.append(ef.astype(jnp.bfloat16))
            s = s + ef
            sq = sq + ef * ef
        e = jnp.concatenate(efs, axis=1)
        fm2 = 0.5 * (jnp.sum(s * s, axis=1) - jnp.sum(sq, axis=1))

        h = e
        for w_ref, b_ref in ((w0_ref, b0_ref), (w1_ref, b1_ref),
                             (w2_ref, b2_ref)):
            h = lax.dot_general(h, w_ref[...], (((1,), (1,)), ((), ())),
                                preferred_element_type=jnp.float32)
            h = jnp.maximum(h + b_ref[...][None, :], 0.0).astype(jnp.bfloat16)
        mlp = jnp.sum(h.astype(jnp.float32) * w3_ref[...], axis=1)

        z = fcsum_ref[...] + bias_ref[0] + b3s_ref[0] + fm2 + mlp
        o_ref[...] = 1.0 / (1.0 + jnp.exp(-z))

    return pl.pallas_call(
        body,
        grid=grid,
        in_specs=[
            pl.BlockSpec((_F, bb, 128), lambda i: (0, i, 0)),
            pl.BlockSpec((bb, 32), lambda i: (i, 0)),
            pl.BlockSpec((bb,), lambda i: (i,)),
            pl.BlockSpec(memory_space=pltpu.SMEM),
            pl.BlockSpec(memory_space=pltpu.SMEM),
            pl.BlockSpec((1024, _FD), lambda i: (0, 0)),
            pl.BlockSpec((1024,), lambda i: (0,)),
            pl.BlockSpec((512, 1024), lambda i: (0, 0)),
            pl.BlockSpec((512,), lambda i: (0,)),
            pl.BlockSpec((256, 512), lambda i: (0, 0)),
            pl.BlockSpec((256,), lambda i: (0,)),
            pl.BlockSpec((1, 256), lambda i: (0, 0)),
        ],
        out_specs=pl.BlockSpec((bb,), lambda i: (i,)),
        out_shape=jax.ShapeDtypeStruct((_B,), jnp.float32),
    )(rows3, xpad, fcsum, bias, b3, W0, b0, W1, b1, W2, b2, W3)


def kernel(x, bias, fc_w, emb_w, W0, b0, W1, b1, W2, b2, W3, b3):
    xi = x.astype(jnp.int32)
    xt3 = xi.reshape(_NW, _BPW, _F).transpose(2, 0, 1)
    xpad = jnp.pad(xi, ((0, 0), (0, 32 - _F)))
    embT = emb_w.T
    fcT = fc_w.T
    w0b = W0.astype(jnp.bfloat16)
    w1b = W1.astype(jnp.bfloat16)
    w2b = W2.astype(jnp.bfloat16)
    pairs, fc_flat = _pairs_table(embT, fcT)
    rows3, fcsum = _sc_gather(xt3, pairs, fc_flat)
    return _tc_fused(rows3, xpad, fcsum, bias,
                     w0b, b0, w1b, b1, w2b, b2, W3, b3)


# speedup vs baseline: 1.1347x; 1.1347x over previous
import functools

import jax
import jax.numpy as jnp
from jax import lax
from jax.experimental import pallas as pl
from jax.experimental.pallas import tpu as pltpu
from jax.experimental.pallas import tpu_sc as plsc

_B = 4096
_F = 26
_D = 64
_FD = _F * _D
_V = 1040000
_TW = 32768
_TBLK = (_V + _TW - 1) // _TW
_QROWS = _TBLK * (_TW // 4)

_NC = 2
_NS = 16
_NW = _NC * _NS
_BPW = _B // _NW
_CHUNK = 128


def _pairs_table(embT, fcT):
    def pack2(hi, lo):
        uh = lax.bitcast_convert_type(hi.astype(jnp.bfloat16), jnp.uint16)
        ul = lax.bitcast_convert_type(lo.astype(jnp.bfloat16), jnp.uint16)
        u = jnp.left_shift(uh.astype(jnp.uint32), jnp.uint32(16)) | ul.astype(jnp.uint32)
        return lax.bitcast_convert_type(u, jnp.float32)

    def body(a_ref, fc_ref, o_ref, o2_ref):
        blk = a_ref[...]
        parts = []
        for t in range(_TW // 512):
            a = blk[:, 512 * t:512 * t + 128]
            b = blk[:, 512 * t + 128:512 * t + 256]
            c = blk[:, 512 * t + 256:512 * t + 384]
            d = blk[:, 512 * t + 384:512 * t + 512]
            parts.append(jnp.concatenate(
                [pack2(a.T, b.T), pack2(c.T, d.T)], axis=1))
        o_ref[...] = jnp.concatenate(parts, axis=0)
        o2_ref[...] = fc_ref[...][0]

    return pl.pallas_call(
        body,
        grid=(_TBLK,),
        in_specs=[pl.BlockSpec((_D, _TW), lambda i: (0, i)),
                  pl.BlockSpec((1, _TW), lambda i: (0, i))],
        out_specs=[pl.BlockSpec((_TW // 4, 128), lambda i: (i, 0)),
                   pl.BlockSpec((_TW,), lambda i: (i,))],
        out_shape=[jax.ShapeDtypeStruct((_QROWS, 128), jnp.float32),
                   jax.ShapeDtypeStruct((_V,), jnp.float32)],
    )(embT, fcT)


def _sc_gather(xt3, pairs, fc_flat):
    mesh = plsc.VectorSubcoreMesh(core_axis_name="c", subcore_axis_name="s")

    @functools.partial(
        pl.kernel,
        mesh=mesh,
        out_type=[jax.ShapeDtypeStruct((_F, _B, 128), jnp.float32),
                  jax.ShapeDtypeStruct((_B,), jnp.float32)],
        scratch_types=[
            pltpu.VMEM((_F, _CHUNK), jnp.int32),
            pltpu.VMEM((_F, _CHUNK), jnp.int32),
            pltpu.VMEM((2, _CHUNK, 128), jnp.float32),
            pltpu.VMEM((_F, _BPW), jnp.float32),
            pltpu.VMEM((_BPW,), jnp.float32),
            pltpu.SemaphoreType.DMA,
            pltpu.SemaphoreType.DMA,
        ],
    )
    def body(xt_hbm, tab_hbm, fc_hbm, rows_out, fc_out,
             idx_v, idxp_v, pairbuf, fcbuf, accv, gsem, fsem):
        wid = lax.axis_index("s") * _NC + lax.axis_index("c")
        pltpu.sync_copy(xt_hbm.at[:, wid], idx_v)
        b0 = wid * _BPW

        def prep_step(j, carry):
            pltpu.async_copy(fc_hbm.at[idx_v.at[j]], fcbuf.at[j], fsem)
            for g in range(_CHUNK // 16):
                v = idx_v[j, pl.ds(g * 16, 16)]
                p = lax.bitwise_or(
                    lax.shift_left(lax.shift_right_logical(v, 9), 7),
                    lax.bitwise_and(v, 127))
                idxp_v[j, pl.ds(g * 16, 16)] = p
            return carry

        lax.fori_loop(0, _F, prep_step, 0)

        pltpu.async_copy(tab_hbm.at[idxp_v.at[0]], pairbuf.at[0], gsem)

        def chunk_step(j, carry):
            slot = lax.rem(j, 2)

            @pl.when(j + 1 < _F)
            def _prefetch():
                pltpu.async_copy(tab_hbm.at[idxp_v.at[j + 1]],
                                 pairbuf.at[lax.rem(j + 1, 2)], gsem)

            pltpu.make_async_copy(tab_hbm.at[idxp_v.at[j]],
                                  pairbuf.at[slot], gsem).wait()
            pltpu.sync_copy(pairbuf.at[slot],
                            rows_out.at[j, pl.ds(b0, _CHUNK)])
            return carry

        lax.fori_loop(0, _F, chunk_step, 0)

        def fc_drain(j, carry):
            pltpu.make_async_copy(fc_hbm.at[idx_v.at[j]], fcbuf.at[j],
                                  fsem).wait()
            return carry

        lax.fori_loop(0, _F, fc_drain, 0)

        for g in range(_BPW // 16):
            acc = jnp.zeros((16,), jnp.float32)
            for f in range(_F):
                acc = acc + fcbuf[f, pl.ds(g * 16, 16)]
            accv[pl.ds(g * 16, 16)] = acc
        pltpu.sync_copy(accv, fc_out.at[pl.ds(wid * _BPW, _BPW)])

    return body(xt3, pairs, fc_flat)


def _tc_fused(rows3, xpad, fcsum, bias, W0, b0, W1, b1, W2, b2, W3, b3):
    bb = 1024
    grid = (_B // bb,)

    def body(p_ref, xp_ref, fcsum_ref, bias_ref, b3s_ref,
             w0_ref, b0_ref, w1_ref, b1_ref, w2_ref, b2_ref, w3_ref, o_ref):
        xp = xp_ref[...]
        efs = []
        s = jnp.zeros((bb, _D), jnp.float32)
        sq = jnp.zeros((bb, _D), jnp.float32)
        for f in range(_F):
            xv = xp[:, f:f + 1]
            h2 = (lax.shift_right_logical(xv, 8) & 1) == 1
            sub = (lax.shift_right_logical(xv, 7) & 1) == 1
            pf = p_ref[f]
            pfh = jnp.where(h2, pf[:, _D:], pf[:, :_D])
            u = lax.bitcast_convert_type(pfh, jnp.uint32)
            su = jnp.where(sub, jnp.left_shift(u, jnp.uint32(16)),
                           u & jnp.uint32(0xFFFF0000))
            ef = lax.bitcast_convert_type(su, jnp.float32)
            efs.append(ef.astype(jnp.bfloat16))
            s = s + ef
            sq = sq + ef * ef
        e = jnp.concatenate(efs, axis=1)
        fm2 = 0.5 * (jnp.sum(s * s, axis=1) - jnp.sum(sq, axis=1))

        h = e
        for w_ref, b_ref in ((w0_ref, b0_ref), (w1_ref, b1_ref),
                             (w2_ref, b2_ref)):
            h = lax.dot_general(h, w_ref[...], (((1,), (1,)), ((), ())),
                                preferred_element_type=jnp.float32)
            h = jnp.maximum(h + b_ref[...][None, :], 0.0).astype(jnp.bfloat16)
        mlp = jnp.sum(h.astype(jnp.float32) * w3_ref[...], axis=1)

        z = fcsum_ref[...] + bias_ref[0] + b3s_ref[0] + fm2 + mlp
        o_ref[...] = 1.0 / (1.0 + jnp.exp(-z))

    return pl.pallas_call(
        body,
        grid=grid,
        in_specs=[
            pl.BlockSpec((_F, bb, 128), lambda i: (0, i, 0)),
            pl.BlockSpec((bb, 32), lambda i: (i, 0)),
            pl.BlockSpec((bb,), lambda i: (i,)),
            pl.BlockSpec(memory_space=pltpu.SMEM),
            pl.BlockSpec(memory_space=pltpu.SMEM),
            pl.BlockSpec((1024, _FD), lambda i: (0, 0)),
            pl.BlockSpec((1024,), lambda i: (0,)),
            pl.BlockSpec((512, 1024), lambda i: (0, 0)),
            pl.BlockSpec((512,), lambda i: (0,)),
            pl.BlockSpec((256, 512), lambda i: (0, 0)),
            pl.BlockSpec((256,), lambda i: (0,)),
            pl.BlockSpec((1, 256), lambda i: (0, 0)),
        ],
        out_specs=pl.BlockSpec((bb,), lambda i: (i,)),
        out_shape=jax.ShapeDtypeStruct((_B,), jnp.float32),
    )(rows3, xpad, fcsum, bias, b3, W0, b0, W1, b1, W2, b2, W3)


def kernel(x, bias, fc_w, emb_w, W0, b0, W1, b1, W2, b2, W3, b3):
    xi = x.astype(jnp.int32)
    xt3 = xi.reshape(_NW, _BPW, _F).transpose(2, 0, 1)
    xpad = jnp.pad(xi, ((0, 0), (0, 32 - _F)))
    embT = emb_w.T
    fcT = fc_w.T
    w0b = W0.astype(jnp.bfloat16)
    w1b = W1.astype(jnp.bfloat16)
    w2b = W2.astype(jnp.bfloat16)
    pairs, fc_flat = _pairs_table(embT, fcT)
    rows3, fcsum = _sc_gather(xt3, pairs, fc_flat)
    return _tc_fused(rows3, xpad, fcsum, bias,
                     w0b, b0, w1b, b1, w2b, b2, W3, b3)


# speedup vs baseline: 1.1757x; 1.0361x over previous
import functools

import jax
import jax.numpy as jnp
from jax import lax
from jax.experimental import pallas as pl
from jax.experimental.pallas import tpu as pltpu
from jax.experimental.pallas import tpu_sc as plsc

_B = 4096
_F = 26
_D = 64
_FD = _F * _D
_V = 1040000
_TW = 32768
_TBLK = (_V + _TW - 1) // _TW
_QROWS = _TBLK * (_TW // 4)

_NC = 2
_NS = 16
_NW = _NC * _NS
_BPW = _B // _NW
_CHUNK = 128


def _pairs_table(embT, fcT):
    def pack2(hi, lo):
        uh = lax.bitcast_convert_type(hi.astype(jnp.bfloat16), jnp.uint16)
        ul = lax.bitcast_convert_type(lo.astype(jnp.bfloat16), jnp.uint16)
        u = jnp.left_shift(uh.astype(jnp.uint32), jnp.uint32(16)) | ul.astype(jnp.uint32)
        return lax.bitcast_convert_type(u, jnp.float32)

    def body(a_ref, fc_ref, o_ref, o2_ref):
        blk = a_ref[...]
        parts = []
        for t in range(_TW // 512):
            a = blk[:, 512 * t:512 * t + 128]
            b = blk[:, 512 * t + 128:512 * t + 256]
            c = blk[:, 512 * t + 256:512 * t + 384]
            d = blk[:, 512 * t + 384:512 * t + 512]
            parts.append(jnp.concatenate(
                [pack2(a.T, b.T), pack2(c.T, d.T)], axis=1))
        o_ref[...] = jnp.concatenate(parts, axis=0)
        o2_ref[...] = fc_ref[...][0]

    return pl.pallas_call(
        body,
        grid=(_TBLK,),
        in_specs=[pl.BlockSpec((_D, _TW), lambda i: (0, i)),
                  pl.BlockSpec((1, _TW), lambda i: (0, i))],
        out_specs=[pl.BlockSpec((_TW // 4, 128), lambda i: (i, 0)),
                   pl.BlockSpec((_TW,), lambda i: (i,))],
        out_shape=[jax.ShapeDtypeStruct((_QROWS, 128), jnp.float32),
                   jax.ShapeDtypeStruct((_V,), jnp.float32)],
    )(embT, fcT)


def _sc_gather(xt3, pairs, fc_flat):
    nb = xt3.shape[1] * xt3.shape[2]
    bpw = nb // _NW
    mesh = plsc.VectorSubcoreMesh(core_axis_name="c", subcore_axis_name="s")

    @functools.partial(
        pl.kernel,
        mesh=mesh,
        out_type=[jax.ShapeDtypeStruct((_F, nb, 128), jnp.float32),
                  jax.ShapeDtypeStruct((nb,), jnp.float32)],
        scratch_types=[
            pltpu.VMEM((_F, bpw), jnp.int32),
            pltpu.VMEM((_F, bpw), jnp.int32),
            pltpu.VMEM((2, bpw, 128), jnp.float32),
            pltpu.VMEM((_F, bpw), jnp.float32),
            pltpu.VMEM((bpw,), jnp.float32),
            pltpu.SemaphoreType.DMA,
            pltpu.SemaphoreType.DMA,
        ],
    )
    def body(xt_hbm, tab_hbm, fc_hbm, rows_out, fc_out,
             idx_v, idxp_v, pairbuf, fcbuf, accv, gsem, fsem):
        wid = lax.axis_index("s") * _NC + lax.axis_index("c")
        pltpu.sync_copy(xt_hbm.at[:, wid], idx_v)
        b0 = wid * bpw

        def prep_step(j, carry):
            pltpu.async_copy(fc_hbm.at[idx_v.at[j]], fcbuf.at[j], fsem)
            for g in range(bpw // 16):
                v = idx_v[j, pl.ds(g * 16, 16)]
                p = lax.bitwise_or(
                    lax.shift_left(lax.shift_right_logical(v, 9), 7),
                    lax.bitwise_and(v, 127))
                idxp_v[j, pl.ds(g * 16, 16)] = p
            return carry

        lax.fori_loop(0, _F, prep_step, 0)

        pltpu.async_copy(tab_hbm.at[idxp_v.at[0]], pairbuf.at[0], gsem)

        def chunk_step(j, carry):
            slot = lax.rem(j, 2)

            @pl.when(j + 1 < _F)
            def _prefetch():
                pltpu.async_copy(tab_hbm.at[idxp_v.at[j + 1]],
                                 pairbuf.at[lax.rem(j + 1, 2)], gsem)

            pltpu.make_async_copy(tab_hbm.at[idxp_v.at[j]],
                                  pairbuf.at[slot], gsem).wait()
            pltpu.sync_copy(pairbuf.at[slot],
                            rows_out.at[j, pl.ds(b0, bpw)])
            return carry

        lax.fori_loop(0, _F, chunk_step, 0)

        def fc_drain(j, carry):
            pltpu.make_async_copy(fc_hbm.at[idx_v.at[j]], fcbuf.at[j],
                                  fsem).wait()
            return carry

        lax.fori_loop(0, _F, fc_drain, 0)

        for g in range(bpw // 16):
            acc = jnp.zeros((16,), jnp.float32)
            for f in range(_F):
                acc = acc + fcbuf[f, pl.ds(g * 16, 16)]
            accv[pl.ds(g * 16, 16)] = acc
        pltpu.sync_copy(accv, fc_out.at[pl.ds(wid * bpw, bpw)])

    return body(xt3, pairs, fc_flat)


def _tc_fused(rows3, xpad, fcsum, bias, W0, b0, W1, b1, W2, b2, W3, b3):
    nb = rows3.shape[1]
    bb = 512
    grid = (nb // bb,)

    def body(p_ref, xp_ref, fcsum_ref, bias_ref, b3s_ref,
             w0_ref, b0_ref, w1_ref, b1_ref, w2_ref, b2_ref, w3_ref, o_ref):
        xp = xp_ref[...]
        efs = []
        s = jnp.zeros((bb, _D), jnp.float32)
        sq = jnp.zeros((bb, _D), jnp.float32)
        for f in range(_F):
            xv = xp[:, f:f + 1]
            h2 = (lax.shift_right_logical(xv, 8) & 1) == 1
            sub = (lax.shift_right_logical(xv, 7) & 1) == 1
            pf = p_ref[f]
            pfh = jnp.where(h2, pf[:, _D:], pf[:, :_D])
            u = lax.bitcast_convert_type(pfh, jnp.uint32)
            su = jnp.where(sub, jnp.left_shift(u, jnp.uint32(16)),
                           u & jnp.uint32(0xFFFF0000))
            ef = lax.bitcast_convert_type(su, jnp.float32)
            efs.append(ef.astype(jnp.bfloat16))
            s = s + ef
            sq = sq + ef * ef
        e = jnp.concatenate(efs, axis=1)
        fm2 = 0.5 * (jnp.sum(s * s, axis=1) - jnp.sum(sq, axis=1))

        h = e
        for w_ref, b_ref in ((w0_ref, b0_ref), (w1_ref, b1_ref),
                             (w2_ref, b2_ref)):
            h = lax.dot_general(h, w_ref[...], (((1,), (1,)), ((), ())),
                                preferred_element_type=jnp.float32)
            h = jnp.maximum(h + b_ref[...][None, :], 0.0).astype(jnp.bfloat16)
        mlp = jnp.sum(h.astype(jnp.float32) * w3_ref[...], axis=1)

        z = fcsum_ref[...] + bias_ref[0] + b3s_ref[0] + fm2 + mlp
        o_ref[...] = 1.0 / (1.0 + jnp.exp(-z))

    return pl.pallas_call(
        body,
        grid=grid,
        in_specs=[
            pl.BlockSpec((_F, bb, 128), lambda i: (0, i, 0)),
            pl.BlockSpec((bb, 32), lambda i: (i, 0)),
            pl.BlockSpec((bb,), lambda i: (i,)),
            pl.BlockSpec(memory_space=pltpu.SMEM),
            pl.BlockSpec(memory_space=pltpu.SMEM),
            pl.BlockSpec((1024, _FD), lambda i: (0, 0)),
            pl.BlockSpec((1024,), lambda i: (0,)),
            pl.BlockSpec((512, 1024), lambda i: (0, 0)),
            pl.BlockSpec((512,), lambda i: (0,)),
            pl.BlockSpec((256, 512), lambda i: (0, 0)),
            pl.BlockSpec((256,), lambda i: (0,)),
            pl.BlockSpec((1, 256), lambda i: (0, 0)),
        ],
        out_specs=pl.BlockSpec((bb,), lambda i: (i,)),
        out_shape=jax.ShapeDtypeStruct((nb,), jnp.float32),
    )(rows3, xpad, fcsum, bias, b3, W0, b0, W1, b1, W2, b2, W3)


def kernel(x, bias, fc_w, emb_w, W0, b0, W1, b1, W2, b2, W3, b3):
    xi = x.astype(jnp.int32)
    hb = _B // 2
    xt4 = xi.reshape(2, _NW, hb // _NW, _F).transpose(0, 3, 1, 2)
    xpad = jnp.pad(xi, ((0, 0), (0, 32 - _F)))
    embT = emb_w.T
    fcT = fc_w.T
    w0b = W0.astype(jnp.bfloat16)
    w1b = W1.astype(jnp.bfloat16)
    w2b = W2.astype(jnp.bfloat16)
    pairs, fc_flat = _pairs_table(embT, fcT)
    outs = []
    for h in range(2):
        rows3, fcsum = _sc_gather(xt4[h], pairs, fc_flat)
        outs.append(_tc_fused(rows3, xpad[h * hb:(h + 1) * hb], fcsum, bias,
                              w0b, b0, w1b, b1, w2b, b2, W3, b3))
    return jnp.concatenate(outs)


# speedup vs baseline: 1.1769x; 1.0011x over previous
import functools

import jax
import jax.numpy as jnp
from jax import lax
from jax.experimental import pallas as pl
from jax.experimental.pallas import tpu as pltpu
from jax.experimental.pallas import tpu_sc as plsc

_B = 4096
_F = 26
_D = 64
_FD = _F * _D
_V = 1040000
_TW = 32768
_TBLK = (_V + _TW - 1) // _TW
_QROWS = _TBLK * (_TW // 4)

_NC = 2
_NS = 16
_NW = _NC * _NS
_BPW = _B // _NW
_CHUNK = 128


def _pairs_table(embT, fcT):
    def pack2(hi, lo):
        uh = lax.bitcast_convert_type(hi.astype(jnp.bfloat16), jnp.uint16)
        ul = lax.bitcast_convert_type(lo.astype(jnp.bfloat16), jnp.uint16)
        u = jnp.left_shift(uh.astype(jnp.uint32), jnp.uint32(16)) | ul.astype(jnp.uint32)
        return lax.bitcast_convert_type(u, jnp.float32)

    def body(a_ref, fc_ref, o_ref, o2_ref):
        blk = a_ref[...]
        parts = []
        for t in range(_TW // 512):
            a = blk[:, 512 * t:512 * t + 128]
            b = blk[:, 512 * t + 128:512 * t + 256]
            c = blk[:, 512 * t + 256:512 * t + 384]
            d = blk[:, 512 * t + 384:512 * t + 512]
            parts.append(jnp.concatenate(
                [pack2(a.T, b.T), pack2(c.T, d.T)], axis=1))
        o_ref[...] = jnp.concatenate(parts, axis=0)
        o2_ref[...] = fc_ref[...][0]

    return pl.pallas_call(
        body,
        grid=(_TBLK,),
        in_specs=[pl.BlockSpec((_D, _TW), lambda i: (0, i)),
                  pl.BlockSpec((1, _TW), lambda i: (0, i))],
        out_specs=[pl.BlockSpec((_TW // 4, 128), lambda i: (i, 0)),
                   pl.BlockSpec((_TW,), lambda i: (i,))],
        out_shape=[jax.ShapeDtypeStruct((_QROWS, 128), jnp.float32),
                   jax.ShapeDtypeStruct((_V,), jnp.float32)],
    )(embT, fcT)


def _sc_gather(xt3, pairs, fc_flat):
    nb = xt3.shape[1] * xt3.shape[2]
    bpw = nb // _NW
    mesh = plsc.VectorSubcoreMesh(core_axis_name="c", subcore_axis_name="s")

    @functools.partial(
        pl.kernel,
        mesh=mesh,
        out_type=[jax.ShapeDtypeStruct((_F, nb, 128), jnp.float32),
                  jax.ShapeDtypeStruct((nb,), jnp.float32)],
        scratch_types=[
            pltpu.VMEM((_F, bpw), jnp.int32),
            pltpu.VMEM((_F, bpw), jnp.int32),
            pltpu.VMEM((2, bpw, 128), jnp.float32),
            pltpu.VMEM((_F, bpw), jnp.float32),
            pltpu.VMEM((bpw,), jnp.float32),
            pltpu.SemaphoreType.DMA,
            pltpu.SemaphoreType.DMA,
        ],
    )
    def body(xt_hbm, tab_hbm, fc_hbm, rows_out, fc_out,
             idx_v, idxp_v, pairbuf, fcbuf, accv, gsem, fsem):
        wid = lax.axis_index("s") * _NC + lax.axis_index("c")
        pltpu.sync_copy(xt_hbm.at[:, wid], idx_v)
        b0 = wid * bpw

        def prep_step(j, carry):
            pltpu.async_copy(fc_hbm.at[idx_v.at[j]], fcbuf.at[j], fsem)
            for g in range(bpw // 16):
                v = idx_v[j, pl.ds(g * 16, 16)]
                p = lax.bitwise_or(
                    lax.shift_left(lax.shift_right_logical(v, 9), 7),
                    lax.bitwise_and(v, 127))
                idxp_v[j, pl.ds(g * 16, 16)] = p
            return carry

        lax.fori_loop(0, _F, prep_step, 0)

        pltpu.async_copy(tab_hbm.at[idxp_v.at[0]], pairbuf.at[0], gsem)

        def chunk_step(j, carry):
            slot = lax.rem(j, 2)

            @pl.when(j + 1 < _F)
            def _prefetch():
                pltpu.async_copy(tab_hbm.at[idxp_v.at[j + 1]],
                                 pairbuf.at[lax.rem(j + 1, 2)], gsem)

            pltpu.make_async_copy(tab_hbm.at[idxp_v.at[j]],
                                  pairbuf.at[slot], gsem).wait()
            pltpu.sync_copy(pairbuf.at[slot],
                            rows_out.at[j, pl.ds(b0, bpw)])
            return carry

        lax.fori_loop(0, _F, chunk_step, 0)

        def fc_drain(j, carry):
            pltpu.make_async_copy(fc_hbm.at[idx_v.at[j]], fcbuf.at[j],
                                  fsem).wait()
            return carry

        lax.fori_loop(0, _F, fc_drain, 0)

        for g in range(bpw // 16):
            acc = jnp.zeros((16,), jnp.float32)
            for f in range(_F):
                acc = acc + fcbuf[f, pl.ds(g * 16, 16)]
            accv[pl.ds(g * 16, 16)] = acc
        pltpu.sync_copy(accv, fc_out.at[pl.ds(wid * bpw, bpw)])

    return body(xt3, pairs, fc_flat)


def _tc_fused(rows3, xpad, fcsum, bias, W0, b0, W1, b1, W2, b2, W3, b3):
    nb = rows3.shape[1]
    bb = 512
    grid = (nb // bb,)

    def body(p_ref, xp_ref, fcsum_ref, bias_ref, b3s_ref,
             w0_ref, b0_ref, w1_ref, b1_ref, w2_ref, b2_ref, w3_ref, o_ref):
        xp = xp_ref[...]
        s = jnp.zeros((bb, _D), jnp.float32)
        sq = jnp.zeros((bb, _D), jnp.float32)
        h0 = jnp.zeros((bb, 1024), jnp.float32)
        prev = None
        for f in range(_F):
            xv = xp[:, f:f + 1]
            h2 = (lax.shift_right_logical(xv, 8) & 1) == 1
            sub = (lax.shift_right_logical(xv, 7) & 1) == 1
            pf = p_ref[f]
            pfh = jnp.where(h2, pf[:, _D:], pf[:, :_D])
            u = lax.bitcast_convert_type(pfh, jnp.uint32)
            su = jnp.where(sub, jnp.left_shift(u, jnp.uint32(16)),
                           u & jnp.uint32(0xFFFF0000))
            ef = lax.bitcast_convert_type(su, jnp.float32)
            s = s + ef
            sq = sq + ef * ef
            if prev is None:
                prev = ef
            else:
                k = f // 2
                e2 = jnp.concatenate([prev, ef], axis=1).astype(jnp.bfloat16)
                w0k = w0_ref[:, pl.ds(k * 128, 128)]
                h0 = h0 + lax.dot_general(
                    e2, w0k, (((1,), (1,)), ((), ())),
                    preferred_element_type=jnp.float32)
                prev = None
        fm2 = 0.5 * (jnp.sum(s * s, axis=1) - jnp.sum(sq, axis=1))

        h = jnp.maximum(h0 + b0_ref[...][None, :], 0.0).astype(jnp.bfloat16)
        for w_ref, b_ref in ((w1_ref, b1_ref), (w2_ref, b2_ref)):
            h = lax.dot_general(h, w_ref[...], (((1,), (1,)), ((), ())),
                                preferred_element_type=jnp.float32)
            h = jnp.maximum(h + b_ref[...][None, :], 0.0).astype(jnp.bfloat16)
        mlp = jnp.sum(h.astype(jnp.float32) * w3_ref[...], axis=1)

        z = fcsum_ref[...] + bias_ref[0] + b3s_ref[0] + fm2 + mlp
        o_ref[...] = 1.0 / (1.0 + jnp.exp(-z))

    return pl.pallas_call(
        body,
        grid=grid,
        in_specs=[
            pl.BlockSpec((_F, bb, 128), lambda i: (0, i, 0)),
            pl.BlockSpec((bb, 32), lambda i: (i, 0)),
            pl.BlockSpec((bb,), lambda i: (i,)),
            pl.BlockSpec(memory_space=pltpu.SMEM),
            pl.BlockSpec(memory_space=pltpu.SMEM),
            pl.BlockSpec((1024, _FD), lambda i: (0, 0)),
            pl.BlockSpec((1024,), lambda i: (0,)),
            pl.BlockSpec((512, 1024), lambda i: (0, 0)),
            pl.BlockSpec((512,), lambda i: (0,)),
            pl.BlockSpec((256, 512), lambda i: (0, 0)),
            pl.BlockSpec((256,), lambda i: (0,)),
            pl.BlockSpec((1, 256), lambda i: (0, 0)),
        ],
        out_specs=pl.BlockSpec((bb,), lambda i: (i,)),
        out_shape=jax.ShapeDtypeStruct((nb,), jnp.float32),
    )(rows3, xpad, fcsum, bias, b3, W0, b0, W1, b1, W2, b2, W3)


def kernel(x, bias, fc_w, emb_w, W0, b0, W1, b1, W2, b2, W3, b3):
    xi = x.astype(jnp.int32)
    hb = _B // 2
    xt4 = xi.reshape(2, _NW, hb // _NW, _F).transpose(0, 3, 1, 2)
    xpad = jnp.pad(xi, ((0, 0), (0, 32 - _F)))
    embT = emb_w.T
    fcT = fc_w.T
    w0b = W0.astype(jnp.bfloat16)
    w1b = W1.astype(jnp.bfloat16)
    w2b = W2.astype(jnp.bfloat16)
    pairs, fc_flat = _pairs_table(embT, fcT)
    outs = []
    for h in range(2):
        rows3, fcsum = _sc_gather(xt4[h], pairs, fc_flat)
        outs.append(_tc_fused(rows3, xpad[h * hb:(h + 1) * hb], fcsum, bias,
                              w0b, b0, w1b, b1, w2b, b2, W3, b3))
    return jnp.concatenate(outs)
